# baseline (speedup 1.0000x reference)
import functools

import jax
import jax.numpy as jnp
from jax import lax
from jax.experimental import pallas as pl
from jax.experimental.pallas import tpu as pltpu
from jax.experimental.pallas import tpu_sc as plsc

N_BASES = 4
L = 2048
BLOCK = 256
NUM_CORES = 2
NUM_SUBCORES = 16
NW = NUM_CORES * NUM_SUBCORES
ROWS_PER_W = L // NW
SC_BLK = 16


def _seq_body(seq_ref, seq_out_ref):
    i = pl.program_id(0)
    si = seq_ref[0, pl.ds(i * BLOCK, BLOCK)]
    sj = seq_ref[0, :]
    for c in range(N_BASES):
        seq_out_ref[c] = jnp.broadcast_to(
            (si[:, None] == c).astype(jnp.float32), (BLOCK, L))
    for c in range(N_BASES):
        seq_out_ref[c + N_BASES] = jnp.broadcast_to(
            (sj[None, :] == c).astype(jnp.float32), (BLOCK, L))


def _seq_out(seq2d):
    return pl.pallas_call(
        _seq_body,
        grid=(L // BLOCK,),
        in_specs=[pl.BlockSpec((1, L), lambda i: (0, 0))],
        out_specs=pl.BlockSpec((2 * N_BASES, BLOCK, L), lambda i: (0, i, 0)),
        out_shape=jax.ShapeDtypeStruct((2 * N_BASES, L, L), jnp.float32),
        compiler_params=pltpu.CompilerParams(
            dimension_semantics=("arbitrary",)),
    )(seq2d)


@functools.partial(
    pl.kernel,
    out_type=jax.ShapeDtypeStruct((1, L, L), jnp.float32),
    mesh=plsc.VectorSubcoreMesh(
        core_axis_name="c", subcore_axis_name="s",
        num_cores=NUM_CORES, num_subcores=NUM_SUBCORES),
    scratch_types=[
        pltpu.VMEM((SC_BLK,), jnp.int32),
        pltpu.VMEM((SC_BLK, L), jnp.float32),
    ],
    compiler_params=pltpu.CompilerParams(needs_layout_passes=False),
)
def _idx_out_sc(pair_hbm, out_hbm, idx_v, buf_v):
    wid = lax.axis_index("s") * NUM_CORES + lax.axis_index("c")
    base = wid * ROWS_PER_W

    zeros16 = jnp.zeros((16,), jnp.float32)
    ones16 = jnp.ones((16,), jnp.float32)
    row_iota = lax.iota(jnp.int32, 16)

    def _zero_row(r, carry):
        def _zero_chunk(k, c):
            buf_v[r, pl.ds(k * 16, 16)] = zeros16
            return c
        return lax.fori_loop(0, L // 16, _zero_chunk, carry)
    lax.fori_loop(0, SC_BLK, _zero_row, 0)

    def _block(b, carry):
        r0 = base + b * SC_BLK
        pltpu.sync_copy(pair_hbm.at[pl.ds(r0, SC_BLK)], idx_v)
        cols = idx_v[...]
        plsc.store_scatter(buf_v, [row_iota, cols], ones16)
        pltpu.sync_copy(buf_v, out_hbm.at[0, pl.ds(r0, SC_BLK)])
        plsc.store_scatter(buf_v, [row_iota, cols], zeros16)
        return carry
    lax.fori_loop(0, ROWS_PER_W // SC_BLK, _block, 0)


def kernel(seq_idx, pair_idx):
    n = seq_idx.shape[0]
    seq_out = _seq_out(seq_idx.reshape(1, n))
    idx_out = _idx_out_sc(pair_idx)
    return (seq_out, idx_out)


# speedup vs baseline: 1.3066x; 1.3066x over previous
import jax
import jax.numpy as jnp
from jax.experimental import pallas as pl
from jax.experimental.pallas import tpu as pltpu

N_BASES = 4
L = 2048
BLOCK = 256


def _body(seq_ref, pair_ref, seq_out_ref, idx_out_ref):
    i = pl.program_id(0)
    si = seq_ref[0, pl.ds(i * BLOCK, BLOCK)]
    sj = seq_ref[0, :]
    pi = pair_ref[0, pl.ds(i * BLOCK, BLOCK)]
    jj = jax.lax.broadcasted_iota(jnp.int32, (BLOCK, L), 1)
    for c in range(N_BASES):
        seq_out_ref[c] = jnp.broadcast_to(
            (si[:, None] == c).astype(jnp.float32), (BLOCK, L))
    for c in range(N_BASES):
        seq_out_ref[c + N_BASES] = jnp.broadcast_to(
            (sj[None, :] == c).astype(jnp.float32), (BLOCK, L))
    idx_out_ref[0] = (pi[:, None] == jj).astype(jnp.float32)


def kernel(seq_idx, pair_idx):
    n = seq_idx.shape[0]
    seq2d = seq_idx.reshape(1, n)
    pair2d = pair_idx.reshape(1, n)
    grid = (n // BLOCK,)
    seq_out, idx_out = pl.pallas_call(
        _body,
        grid=grid,
        in_specs=[
            pl.BlockSpec((1, n), lambda i: (0, 0)),
            pl.BlockSpec((1, n), lambda i: (0, 0)),
        ],
        out_specs=[
            pl.BlockSpec((2 * N_BASES, BLOCK, n), lambda i: (0, i, 0)),
            pl.BlockSpec((1, BLOCK, n), lambda i: (0, i, 0)),
        ],
        out_shape=[
            jax.ShapeDtypeStruct((2 * N_BASES, n, n), jnp.float32),
            jax.ShapeDtypeStruct((1, n, n), jnp.float32),
        ],
        compiler_params=pltpu.CompilerParams(
            dimension_semantics=("arbitrary",)),
    )(seq2d, pair2d)
    return (seq_out, idx_out)


# speedup vs baseline: 1.3347x; 1.0215x over previous
import jax
import jax.numpy as jnp
from jax.experimental import pallas as pl
from jax.experimental.pallas import tpu as pltpu

N_BASES = 4
L = 2048
BLOCK = 128


def _body(seq_ref, pair_ref, seq_out_ref, idx_out_ref):
    i = pl.program_id(0)
    si = seq_ref[0, pl.ds(i * BLOCK, BLOCK)]
    sj = seq_ref[0, :]
    pi = pair_ref[0, pl.ds(i * BLOCK, BLOCK)]
    jj = jax.lax.broadcasted_iota(jnp.int32, (BLOCK, L), 1)
    for c in range(N_BASES):
        seq_out_ref[c] = jnp.broadcast_to(
            (si[:, None] == c).astype(jnp.float32), (BLOCK, L))
    for c in range(N_BASES):
        seq_out_ref[c + N_BASES] = jnp.broadcast_to(
            (sj[None, :] == c).astype(jnp.float32), (BLOCK, L))
    idx_out_ref[0] = (pi[:, None] == jj).astype(jnp.float32)


def kernel(seq_idx, pair_idx):
    n = seq_idx.shape[0]
    seq2d = seq_idx.reshape(1, n)
    pair2d = pair_idx.reshape(1, n)
    grid = (n // BLOCK,)
    seq_out, idx_out = pl.pallas_call(
        _body,
        grid=grid,
        in_specs=[
            pl.BlockSpec((1, n), lambda i: (0, 0)),
            pl.BlockSpec((1, n), lambda i: (0, 0)),
        ],
        out_specs=[
            pl.BlockSpec((2 * N_BASES, BLOCK, n), lambda i: (0, i, 0)),
            pl.BlockSpec((1, BLOCK, n), lambda i: (0, i, 0)),
        ],
        out_shape=[
            jax.ShapeDtypeStruct((2 * N_BASES, n, n), jnp.float32),
            jax.ShapeDtypeStruct((1, n, n), jnp.float32),
        ],
        compiler_params=pltpu.CompilerParams(
            dimension_semantics=("arbitrary",)),
    )(seq2d, pair2d)
    return (seq_out, idx_out)
